# baseline (speedup 1.0000x reference)
import functools

import jax
import jax.numpy as jnp
from jax import lax
from jax.experimental import pallas as pl
from jax.experimental.pallas import tpu as pltpu
from jax.experimental.pallas import tpu_sc as plsc

B, L = 1024, 200
N = B * L
VOCAB = 1000000
DID, DCAT = 64, 32
H = 128
MAX_LAG_MIN = 1440.0
MAX_ELAPSED = 300.0

_NC, _NS = 2, 16
_NW = _NC * _NS
_CHUNK = 128
_ROWS_PW = N // _NW
_CHUNKS_PW = _ROWS_PW // _CHUNK

_MESH = dict(core_axis_name="c", subcore_axis_name="s",
             num_cores=_NC, num_subcores=_NS)


def _make_sc_gather_body(width):
    def body(tab, idx, out, idxv, rows_a, rows_b, sem_a, sem_b):
        wid = lax.axis_index("s") * _NC + lax.axis_index("c")
        base = wid * _ROWS_PW
        pltpu.sync_copy(idx.at[pl.ds(base, _ROWS_PW)], idxv)

        def gather(k, buf, sem):
            return pltpu.make_async_copy(
                tab.at[idxv.at[pl.ds(k * _CHUNK, _CHUNK)]], buf, sem)

        def writeback(k, buf):
            pltpu.sync_copy(
                buf, out.at[pl.ds(base + k * _CHUNK, _CHUNK),
                            pl.ds(0, width)])

        gather(0, rows_a, sem_a).start()

        def step(k2, carry):
            k = 2 * k2
            gather(k + 1, rows_b, sem_b).start()
            gather(k, rows_a, sem_a).wait()
            writeback(k, rows_a)

            @pl.when(k + 2 < _CHUNKS_PW)
            def _():
                gather(k + 2, rows_a, sem_a).start()

            gather(k + 1, rows_b, sem_b).wait()
            writeback(k + 1, rows_b)
            return carry

        lax.fori_loop(0, _CHUNKS_PW // 2, step, 0)

    return body


@functools.cache
def _get_sc_gather(width, tiled, dtype):
    return pl.kernel(
        _make_sc_gather_body(width),
        out_type=jax.ShapeDtypeStruct((N, H), dtype),
        mesh=plsc.VectorSubcoreMesh(**_MESH),
        scratch_types=[
            pltpu.VMEM((_ROWS_PW,), jnp.int32),
            pltpu.VMEM((_CHUNK, width), dtype),
            pltpu.VMEM((_CHUNK, width), dtype),
            pltpu.SemaphoreType.DMA,
            pltpu.SemaphoreType.DMA,
        ],
        compiler_params=pltpu.CompilerParams(use_tc_tiling_on_sc=tiled),
    )


def _numeric_feats(ts, el):
    rows, cols = ts.shape
    neg = jnp.int32(-(2 ** 31))
    prev = jnp.concatenate([ts[:1, :], ts[:-1, :]], axis=0)
    row = lax.broadcasted_iota(jnp.int32, ts.shape, 0)
    d = jnp.where(row == 0, ts, jnp.where(ts != prev, prev, neg))
    k = 1
    while k < rows:
        shifted = jnp.concatenate(
            [jnp.full((k, cols), neg, jnp.int32), d[:-k, :]], axis=0)
        d = jnp.maximum(d, shifted)
        k *= 2
    lag = (ts - d).astype(jnp.float32) / (1000.0 * 60.0)
    x0 = jnp.log1p(jnp.clip(lag, 0.0, MAX_LAG_MIN))
    x1 = jnp.clip(el, 0.0, MAX_ELAPSED)
    return x0, x1


def _stats_body(ts_ref, el_ref, st_ref):
    x0, x1 = _numeric_feats(ts_ref[...], el_ref[...])
    m0 = jnp.mean(x0)
    v0 = jnp.mean((x0 - m0) ** 2)
    m1 = jnp.mean(x1)
    v1 = jnp.mean((x1 - m1) ** 2)
    st_ref[...] = jnp.concatenate(
        [m0.reshape(1, 1), v0.reshape(1, 1),
         m1.reshape(1, 1), v1.reshape(1, 1)], axis=1)


_stats = pl.pallas_call(
    _stats_body,
    out_shape=jax.ShapeDtypeStruct((1, 4), jnp.float32),
)


_RB = 16
_TBLK = _RB * L
_STEPS = B // _RB


def _main_body(gid_ref, gcat_ref, ts_ref, el_ref, mid_ref, s_ref, out_ref):
    x0, x1 = _numeric_feats(ts_ref[0], el_ref[0])
    x0_parts = []
    x1_parts = []
    for r in range(_RB):
        x0_parts.append(x0[:, r:r + 1])
        x1_parts.append(x1[:, r:r + 1])
    x0v = jnp.concatenate(x0_parts, axis=0)
    x1v = jnp.concatenate(x1_parts, axis=0)

    s = s_ref[...]
    acc = jnp.dot(gid_ref[...][:, :DID], mid_ref[...],
                  preferred_element_type=jnp.float32)
    acc = acc + gcat_ref[...].astype(jnp.float32)
    acc = acc + x0v * s[0:1, :] + x1v * s[1:2, :] + s[2:3, :]
    mu = jnp.mean(acc, axis=1, keepdims=True)
    xc = acc - mu
    var = jnp.mean(xc * xc, axis=1, keepdims=True)
    out_ref[...] = xc * lax.rsqrt(var + 1e-12) * s[3:4, :] + s[4:5, :]


_main = pl.pallas_call(
    _main_body,
    grid=(_STEPS,),
    in_specs=[
        pl.BlockSpec((_TBLK, H), lambda i: (i, 0)),
        pl.BlockSpec((_TBLK, H), lambda i: (i, 0)),
        pl.BlockSpec((1, L, _RB), lambda i: (i, 0, 0)),
        pl.BlockSpec((1, L, _RB), lambda i: (i, 0, 0)),
        pl.BlockSpec((DID, H), lambda i: (0, 0)),
        pl.BlockSpec((8, H), lambda i: (0, 0)),
    ],
    out_specs=pl.BlockSpec((_TBLK, H), lambda i: (i, 0)),
    out_shape=jax.ShapeDtypeStruct((N, H), jnp.float32),
)


def _to_blocks_t(x):
    return x.reshape(_STEPS, _RB, L).transpose(0, 2, 1)


def kernel(input_ids, category_ids, response_ids, timestamp, elapsed_time,
           resp_emb, id_emb, cat_emb, bn_gamma, bn_beta, num_W, num_b,
           content_W, content_b, resp_W, resp_b, lin_W, lin_b, ln_g, ln_b):
    L1 = lin_W[:, :H].T
    L2 = lin_W[:, H:].T
    m_id = content_W[:, :DID].T @ L1
    m_cat = content_W[:, DID:].T @ L1
    t_cat = cat_emb @ m_cat
    t_resp = resp_emb @ (resp_W[:, :16].T @ L2)
    t_cat4 = (t_cat[:, None, :] + t_resp[None, :, :]).reshape(4 * 1000, H)

    idx_id = input_ids.astype(jnp.int32).reshape(N)
    idx_c4 = (category_ids.astype(jnp.int32) * 4
              + response_ids.astype(jnp.int32)).reshape(N)
    gid = _get_sc_gather(DID, False, jnp.float32)(id_emb, idx_id)
    gcat = _get_sc_gather(H, False, jnp.bfloat16)(
        t_cat4.astype(jnp.bfloat16), idx_c4)

    stats = _stats(timestamp.T, elapsed_time.T)

    x32 = resp_W[:, 16:].T @ L2
    m_num = num_W.T @ x32
    bias = content_b @ L1 + num_b @ x32 + resp_b @ L2 + lin_b

    mean = stats[0, 0::2]
    var = stats[0, 1::2]
    sc = bn_gamma / jnp.sqrt(var + 1e-5)
    tc = bn_beta - mean * sc
    m_num_eff = m_num * sc[:, None]
    bias_f = bias + tc @ m_num

    s = (jnp.zeros((8, H), jnp.float32)
         .at[0:2].set(m_num_eff)
         .at[2].set(bias_f)
         .at[3].set(ln_g)
         .at[4].set(ln_b))

    out = _main(gid, gcat,
                _to_blocks_t(timestamp),
                _to_blocks_t(elapsed_time),
                m_id, s)
    return out.reshape(B, L, H)


# speedup vs baseline: 1.1906x; 1.1906x over previous
import functools

import jax
import jax.numpy as jnp
from jax import lax
from jax.experimental import pallas as pl
from jax.experimental.pallas import tpu as pltpu
from jax.experimental.pallas import tpu_sc as plsc

B, L = 1024, 200
N = B * L
VOCAB = 1000000
DID, DCAT = 64, 32
H = 128
MAX_LAG_MIN = 1440.0
MAX_ELAPSED = 300.0

_NC, _NS = 2, 16
_NW = _NC * _NS
_CHUNK = 128
_ROWS_PW = N // _NW
_CHUNKS_PW = _ROWS_PW // _CHUNK

_MESH = dict(core_axis_name="c", subcore_axis_name="s",
             num_cores=_NC, num_subcores=_NS)


def _make_sc_gather_body(width):
    def body(tab, idx, out, idxv, rows_a, rows_b, sem_a, sem_b):
        wid = lax.axis_index("s") * _NC + lax.axis_index("c")
        base = wid * _ROWS_PW
        pltpu.sync_copy(idx.at[pl.ds(base, _ROWS_PW)], idxv)

        def gather(k, buf, sem):
            return pltpu.make_async_copy(
                tab.at[idxv.at[pl.ds(k * _CHUNK, _CHUNK)]], buf, sem)

        def writeback(k, buf):
            pltpu.sync_copy(
                buf, out.at[pl.ds(base + k * _CHUNK, _CHUNK),
                            pl.ds(0, width)])

        gather(0, rows_a, sem_a).start()

        def step(k2, carry):
            k = 2 * k2
            gather(k + 1, rows_b, sem_b).start()
            gather(k, rows_a, sem_a).wait()
            writeback(k, rows_a)

            @pl.when(k + 2 < _CHUNKS_PW)
            def _():
                gather(k + 2, rows_a, sem_a).start()

            gather(k + 1, rows_b, sem_b).wait()
            writeback(k + 1, rows_b)
            return carry

        lax.fori_loop(0, _CHUNKS_PW // 2, step, 0)

    return body


@functools.cache
def _get_sc_gather(width, tiled, dtype):
    return pl.kernel(
        _make_sc_gather_body(width),
        out_type=jax.ShapeDtypeStruct((N, H), dtype),
        mesh=plsc.VectorSubcoreMesh(**_MESH),
        scratch_types=[
            pltpu.VMEM((_ROWS_PW,), jnp.int32),
            pltpu.VMEM((_CHUNK, width), dtype),
            pltpu.VMEM((_CHUNK, width), dtype),
            pltpu.SemaphoreType.DMA,
            pltpu.SemaphoreType.DMA,
        ],
        compiler_params=pltpu.CompilerParams(use_tc_tiling_on_sc=tiled),
    )


def _numeric_feats(ts, el):
    rows, cols = ts.shape
    neg = jnp.int32(-(2 ** 31))
    prev = jnp.concatenate([ts[:1, :], ts[:-1, :]], axis=0)
    row = lax.broadcasted_iota(jnp.int32, ts.shape, 0)
    d = jnp.where(row == 0, ts, jnp.where(ts != prev, prev, neg))
    k = 1
    while k < rows:
        shifted = jnp.concatenate(
            [jnp.full((k, cols), neg, jnp.int32), d[:-k, :]], axis=0)
        d = jnp.maximum(d, shifted)
        k *= 2
    lag = (ts - d).astype(jnp.float32) / (1000.0 * 60.0)
    x0 = jnp.log1p(jnp.clip(lag, 0.0, MAX_LAG_MIN))
    x1 = jnp.clip(el, 0.0, MAX_ELAPSED)
    return x0, x1


def _stats_body(ts_ref, el_ref, st_ref):
    x0, x1 = _numeric_feats(ts_ref[...], el_ref[...])
    m0 = jnp.mean(x0)
    v0 = jnp.mean((x0 - m0) ** 2)
    m1 = jnp.mean(x1)
    v1 = jnp.mean((x1 - m1) ** 2)
    st_ref[...] = jnp.concatenate(
        [m0.reshape(1, 1), v0.reshape(1, 1),
         m1.reshape(1, 1), v1.reshape(1, 1)], axis=1)


_stats = pl.pallas_call(
    _stats_body,
    out_shape=jax.ShapeDtypeStruct((1, 4), jnp.float32),
)


_RB = 16
_TBLK = _RB * L
_STEPS = B // _RB


def _main_body(gid_ref, gcat_ref, ts_ref, el_ref, mid_ref, s_ref, out_ref):
    x0, x1 = _numeric_feats(ts_ref[0], el_ref[0])
    x0_parts = []
    x1_parts = []
    for r in range(_RB):
        x0_parts.append(x0[:, r:r + 1])
        x1_parts.append(x1[:, r:r + 1])
    x0v = jnp.concatenate(x0_parts, axis=0)
    x1v = jnp.concatenate(x1_parts, axis=0)

    s = s_ref[...]
    acc = jnp.dot(gid_ref[...][:, :DID], mid_ref[...],
                  preferred_element_type=jnp.float32)
    acc = acc + gcat_ref[...]
    acc = acc + x0v * s[0:1, :] + x1v * s[1:2, :] + s[2:3, :]
    mu = jnp.mean(acc, axis=1, keepdims=True)
    xc = acc - mu
    var = jnp.mean(xc * xc, axis=1, keepdims=True)
    out_ref[...] = xc * lax.rsqrt(var + 1e-12) * s[3:4, :] + s[4:5, :]


_main = pl.pallas_call(
    _main_body,
    grid=(_STEPS,),
    in_specs=[
        pl.BlockSpec((_TBLK, H), lambda i: (i, 0)),
        pl.BlockSpec((_TBLK, H), lambda i: (i, 0)),
        pl.BlockSpec((1, L, _RB), lambda i: (i, 0, 0)),
        pl.BlockSpec((1, L, _RB), lambda i: (i, 0, 0)),
        pl.BlockSpec((DID, H), lambda i: (0, 0)),
        pl.BlockSpec((8, H), lambda i: (0, 0)),
    ],
    out_specs=pl.BlockSpec((_TBLK, H), lambda i: (i, 0)),
    out_shape=jax.ShapeDtypeStruct((N, H), jnp.float32),
)


def _to_blocks_t(x):
    return x.reshape(_STEPS, _RB, L).transpose(0, 2, 1)


def kernel(input_ids, category_ids, response_ids, timestamp, elapsed_time,
           resp_emb, id_emb, cat_emb, bn_gamma, bn_beta, num_W, num_b,
           content_W, content_b, resp_W, resp_b, lin_W, lin_b, ln_g, ln_b):
    L1 = lin_W[:, :H].T
    L2 = lin_W[:, H:].T
    m_id = content_W[:, :DID].T @ L1
    m_cat = content_W[:, DID:].T @ L1
    t_cat = cat_emb @ m_cat
    t_resp = resp_emb @ (resp_W[:, :16].T @ L2)
    t_cat4 = (t_cat[:, None, :] + t_resp[None, :, :]).reshape(4 * 1000, H)

    idx_id = input_ids.astype(jnp.int32).reshape(N)
    idx_c4 = (category_ids.astype(jnp.int32) * 4
              + response_ids.astype(jnp.int32)).reshape(N)
    gid = _get_sc_gather(DID, False, jnp.float32)(id_emb, idx_id)
    gcat = _get_sc_gather(H, True, jnp.float32)(t_cat4, idx_c4)

    stats = _stats(timestamp.T, elapsed_time.T)

    x32 = resp_W[:, 16:].T @ L2
    m_num = num_W.T @ x32
    bias = content_b @ L1 + num_b @ x32 + resp_b @ L2 + lin_b

    mean = stats[0, 0::2]
    var = stats[0, 1::2]
    sc = bn_gamma / jnp.sqrt(var + 1e-5)
    tc = bn_beta - mean * sc
    m_num_eff = m_num * sc[:, None]
    bias_f = bias + tc @ m_num

    s = (jnp.zeros((8, H), jnp.float32)
         .at[0:2].set(m_num_eff)
         .at[2].set(bias_f)
         .at[3].set(ln_g)
         .at[4].set(ln_b))

    out = _main(gid, gcat,
                _to_blocks_t(timestamp),
                _to_blocks_t(elapsed_time),
                m_id, s)
    return out.reshape(B, L, H)


# speedup vs baseline: 1.1927x; 1.0018x over previous
import functools

import jax
import jax.numpy as jnp
from jax import lax
from jax.experimental import pallas as pl
from jax.experimental.pallas import tpu as pltpu
from jax.experimental.pallas import tpu_sc as plsc

B, L = 1024, 200
N = B * L
VOCAB = 1000000
DID, DCAT = 64, 32
H = 128
MAX_LAG_MIN = 1440.0
MAX_ELAPSED = 300.0

_NC, _NS = 2, 16
_NW = _NC * _NS
_CHUNK = 128
_ROWS_PW = N // _NW
_CHUNKS_PW = _ROWS_PW // _CHUNK

_MESH = dict(core_axis_name="c", subcore_axis_name="s",
             num_cores=_NC, num_subcores=_NS)


def _make_sc_gather_body(width):
    def body(tab, idx, out, idxv, rows_a, rows_b, sem_a, sem_b):
        wid = lax.axis_index("s") * _NC + lax.axis_index("c")
        base = wid * _ROWS_PW
        pltpu.sync_copy(idx.at[pl.ds(base, _ROWS_PW)], idxv)

        def gather(k, buf, sem):
            return pltpu.make_async_copy(
                tab.at[idxv.at[pl.ds(k * _CHUNK, _CHUNK)]], buf, sem)

        def writeback(k, buf):
            pltpu.sync_copy(
                buf, out.at[pl.ds(base + k * _CHUNK, _CHUNK),
                            pl.ds(0, width)])

        gather(0, rows_a, sem_a).start()

        def step(k2, carry):
            k = 2 * k2
            gather(k + 1, rows_b, sem_b).start()
            gather(k, rows_a, sem_a).wait()
            writeback(k, rows_a)

            @pl.when(k + 2 < _CHUNKS_PW)
            def _():
                gather(k + 2, rows_a, sem_a).start()

            gather(k + 1, rows_b, sem_b).wait()
            writeback(k + 1, rows_b)
            return carry

        lax.fori_loop(0, _CHUNKS_PW // 2, step, 0)

    return body


@functools.cache
def _get_sc_gather(width, tiled, dtype):
    return pl.kernel(
        _make_sc_gather_body(width),
        out_type=jax.ShapeDtypeStruct((N, H), dtype),
        mesh=plsc.VectorSubcoreMesh(**_MESH),
        scratch_types=[
            pltpu.VMEM((_ROWS_PW,), jnp.int32),
            pltpu.VMEM((_CHUNK, width), dtype),
            pltpu.VMEM((_CHUNK, width), dtype),
            pltpu.SemaphoreType.DMA,
            pltpu.SemaphoreType.DMA,
        ],
        compiler_params=pltpu.CompilerParams(use_tc_tiling_on_sc=tiled),
    )


def _numeric_feats(ts, el):
    rows, cols = ts.shape
    neg = jnp.int32(-(2 ** 31))
    prev = jnp.concatenate([ts[:1, :], ts[:-1, :]], axis=0)
    row = lax.broadcasted_iota(jnp.int32, ts.shape, 0)
    d = jnp.where(row == 0, ts, jnp.where(ts != prev, prev, neg))
    k = 1
    while k < rows:
        shifted = jnp.concatenate(
            [jnp.full((k, cols), neg, jnp.int32), d[:-k, :]], axis=0)
        d = jnp.maximum(d, shifted)
        k *= 2
    lag = (ts - d).astype(jnp.float32) / (1000.0 * 60.0)
    x0 = jnp.log1p(jnp.clip(lag, 0.0, MAX_LAG_MIN))
    x1 = jnp.clip(el, 0.0, MAX_ELAPSED)
    return x0, x1


def _stats_body(ts_ref, el_ref, st_ref):
    x0, x1 = _numeric_feats(ts_ref[...], el_ref[...])
    m0 = jnp.mean(x0)
    v0 = jnp.mean((x0 - m0) ** 2)
    m1 = jnp.mean(x1)
    v1 = jnp.mean((x1 - m1) ** 2)
    st_ref[...] = jnp.concatenate(
        [m0.reshape(1, 1), v0.reshape(1, 1),
         m1.reshape(1, 1), v1.reshape(1, 1)], axis=1)


_stats = pl.pallas_call(
    _stats_body,
    out_shape=jax.ShapeDtypeStruct((1, 4), jnp.float32),
)


_RB = 16
_TBLK = _RB * L
_STEPS = B // _RB


def _main_body(gid_ref, gcat_ref, ts_ref, el_ref, mid_ref, s_ref, out_ref):
    x0, x1 = _numeric_feats(ts_ref[0], el_ref[0])
    x0_parts = []
    x1_parts = []
    for r in range(_RB):
        x0_parts.append(x0[:, r:r + 1])
        x1_parts.append(x1[:, r:r + 1])
    x0v = jnp.concatenate(x0_parts, axis=0)
    x1v = jnp.concatenate(x1_parts, axis=0)

    s = s_ref[...]
    acc = jnp.dot(gid_ref[...][:, :DID], mid_ref[...],
                  preferred_element_type=jnp.float32)
    acc = acc + gcat_ref[...]
    acc = acc + x0v * s[0:1, :] + x1v * s[1:2, :] + s[2:3, :]
    mu = jnp.mean(acc, axis=1, keepdims=True)
    xc = acc - mu
    var = jnp.mean(xc * xc, axis=1, keepdims=True)
    out_ref[...] = xc * lax.rsqrt(var + 1e-12) * s[3:4, :] + s[4:5, :]


_main = pl.pallas_call(
    _main_body,
    grid=(_STEPS,),
    in_specs=[
        pl.BlockSpec((_TBLK, H), lambda i: (i, 0)),
        pl.BlockSpec((_TBLK, H), lambda i: (i, 0)),
        pl.BlockSpec((1, L, _RB), lambda i: (i, 0, 0)),
        pl.BlockSpec((1, L, _RB), lambda i: (i, 0, 0)),
        pl.BlockSpec((DID, H), lambda i: (0, 0)),
        pl.BlockSpec((8, H), lambda i: (0, 0)),
    ],
    out_specs=pl.BlockSpec((_TBLK, H), lambda i: (i, 0)),
    out_shape=jax.ShapeDtypeStruct((N, H), jnp.float32),
)


def _to_blocks_t(x):
    return x.reshape(_STEPS, _RB, L).transpose(0, 2, 1)


def kernel(input_ids, category_ids, response_ids, timestamp, elapsed_time,
           resp_emb, id_emb, cat_emb, bn_gamma, bn_beta, num_W, num_b,
           content_W, content_b, resp_W, resp_b, lin_W, lin_b, ln_g, ln_b):
    L1 = lin_W[:, :H].T
    L2 = lin_W[:, H:].T
    m_id = content_W[:, :DID].T @ L1
    m_cat = content_W[:, DID:].T @ L1
    t_cat = cat_emb @ m_cat
    t_resp = resp_emb @ (resp_W[:, :16].T @ L2)
    t_cat4 = (t_cat[:, None, :] + t_resp[None, :, :]).reshape(4 * 1000, H)

    idx_id = input_ids.astype(jnp.int32).reshape(N)
    idx_c4 = (category_ids.astype(jnp.int32) * 4
              + response_ids.astype(jnp.int32)).reshape(N)
    id_flat = lax.optimization_barrier(id_emb.reshape(-1))
    gid = _get_sc_gather(DID, False, jnp.float32)(
        id_flat.reshape(VOCAB, DID), idx_id)
    gcat = _get_sc_gather(H, True, jnp.float32)(t_cat4, idx_c4)

    stats = _stats(timestamp.T, elapsed_time.T)

    x32 = resp_W[:, 16:].T @ L2
    m_num = num_W.T @ x32
    bias = content_b @ L1 + num_b @ x32 + resp_b @ L2 + lin_b

    mean = stats[0, 0::2]
    var = stats[0, 1::2]
    sc = bn_gamma / jnp.sqrt(var + 1e-5)
    tc = bn_beta - mean * sc
    m_num_eff = m_num * sc[:, None]
    bias_f = bias + tc @ m_num

    s = (jnp.zeros((8, H), jnp.float32)
         .at[0:2].set(m_num_eff)
         .at[2].set(bias_f)
         .at[3].set(ln_g)
         .at[4].set(ln_b))

    out = _main(gid, gcat,
                _to_blocks_t(timestamp),
                _to_blocks_t(elapsed_time),
                m_id, s)
    return out.reshape(B, L, H)
